# baseline (speedup 1.0000x reference)
import functools

import numpy as np
import jax
import jax.numpy as jnp
from jax import lax
from jax.experimental import pallas as pl
from jax.experimental.pallas import tpu as pltpu
from jax.experimental.pallas import tpu_sc as plsc

F = 26
V = 100000
D = 32
B = 16384
CD = 64
XR = CD // D
OR = F + XR

NC = 2
NS = 16
NW = NC * NS
BW = B // NW
CB = 64
NCHUNK = BW // CB
ROWS = CB * F
NT = ROWS // 128

def _body(x2, vals, tab, out, vals_v, gidx_v, oidx_v, xidx_v, rows_v, x_v,
          patg_v, pato_v, patx_v, sem):
    wid = lax.axis_index("s") * NC + lax.axis_index("c")

    idx16 = lax.iota(jnp.int32, 16)

    def _splat(c):
        return jnp.full((16,), c, jnp.int32)

    for j in range(13):
        q = idx16 + j * 16
        f = lax.rem(q, _splat(F))
        patg_v[j, :] = f * V
        pato_v[j, :] = lax.div(q, _splat(F)) * OR + f
    patx_v[:] = lax.div(idx16, _splat(XR)) * OR + F + lax.rem(idx16, _splat(XR))

    def chunk_body(c, carry):
        r0 = wid * BW + c * CB
        pltpu.sync_copy(vals.at[pl.ds(r0 * F, ROWS)], vals_v)
        pltpu.sync_copy(x2.at[pl.ds(r0 * XR, CB * XR)], x_v)
        base_out = r0 * OR
        for g in range(8):
            goff = base_out + g * 208 // F * OR
            for j in range(13):
                i = g * 13 + j
                vv = vals_v[pl.ds(i * 16, 16)]
                gidx_v[i // 8, pl.ds((i % 8) * 16, 16)] = vv + patg_v[j, :]
                oidx_v[i // 8, pl.ds((i % 8) * 16, 16)] = pato_v[j, :] + goff
        for k in range(CB * XR // 16):
            xidx_v[0, pl.ds(k * 16, 16)] = patx_v[:] + (base_out + k * 8 * OR)
        cps = [
            pltpu.async_copy(
                tab.at[gidx_v.at[t]], rows_v.at[pl.ds(t * 128, 128)], sem
            )
            for t in range(NT)
        ]
        for cp in cps:
            cp.wait()
        cps = [
            pltpu.async_copy(
                rows_v.at[pl.ds(t * 128, 128)], out.at[oidx_v.at[t]], sem
            )
            for t in range(NT)
        ]
        cps.append(pltpu.async_copy(x_v, out.at[xidx_v.at[0]], sem))
        for cp in cps:
            cp.wait()
        return carry

    lax.fori_loop(0, NCHUNK, chunk_body, 0)


@functools.partial(jax.jit, donate_argnums=())
def _run(x2, vals, tab):
    mesh = plsc.VectorSubcoreMesh(core_axis_name="c", subcore_axis_name="s")
    kern = functools.partial(
        pl.kernel,
        out_type=jax.ShapeDtypeStruct((B * OR, D), jnp.float32),
        mesh=mesh,
        compiler_params=pltpu.CompilerParams(use_tc_tiling_on_sc=False),
        scratch_types=[
            pltpu.VMEM((ROWS,), jnp.int32),
            pltpu.VMEM((NT, 128), jnp.int32),
            pltpu.VMEM((NT, 128), jnp.int32),
            pltpu.VMEM((1, 128), jnp.int32),
            pltpu.VMEM((ROWS, D), jnp.float32),
            pltpu.VMEM((CB * XR, D), jnp.float32),
            pltpu.VMEM((13, 16), jnp.int32),
            pltpu.VMEM((13, 16), jnp.int32),
            pltpu.VMEM((16,), jnp.int32),
            pltpu.SemaphoreType.DMA,
        ],
    )(_body)
    return kern(x2, vals, tab)


def kernel(x, values, tables):
    x2 = x.reshape(B * XR, D)
    vals = values.reshape(B * F)
    tab = tables.reshape(F * V, D)
    out = _run(x2, vals, tab)
    return out.reshape(B, F * D + CD)
